# baseline (speedup 1.0000x reference)
import jax
import jax.numpy as jnp
from jax import lax
from jax.experimental import pallas as pl

_DEPTH = 1000
_ROWS = 1024
_COLS = 50
_BLK = 8


def _onehot_body(x_ref, o_ref):
    xv = x_ref[...]
    col = lax.broadcasted_iota(jnp.int32, (_BLK, _COLS, _DEPTH), 2)
    o_ref[...] = (col == xv[:, :, None]).astype(jnp.float32)


def kernel(x):
    x = x.astype(jnp.int32)
    return pl.pallas_call(
        _onehot_body,
        grid=(_ROWS // _BLK,),
        in_specs=[pl.BlockSpec((_BLK, _COLS), lambda i: (i, 0))],
        out_specs=pl.BlockSpec((_BLK, _COLS, _DEPTH), lambda i: (i, 0, 0)),
        out_shape=jax.ShapeDtypeStruct((_ROWS, _COLS, _DEPTH), jnp.float32),
    )(x)


# speedup vs baseline: 1.0199x; 1.0199x over previous
import functools

import jax
import jax.numpy as jnp
from jax import lax
from jax.experimental import pallas as pl
from jax.experimental.pallas import tpu as pltpu
from jax.experimental.pallas import tpu_sc as plsc

_B, _S, _D = 1024, 50, 1000
_SP = 64
_NW = 32
_BPW = _B // _NW


def _onehot_sc(x_hbm, out_hbm, xbuf, stage, sem0, sem1):
    cid = lax.axis_index("c")
    sid = lax.axis_index("s")
    wid = sid * 2 + cid
    base = wid * _BPW

    sems = (sem0, sem1)
    i16 = lax.broadcasted_iota(jnp.int32, (16,), 0)
    zeros = jnp.zeros((16,), jnp.float32)

    pltpu.sync_copy(x_hbm.at[pl.ds(base, _BPW)], xbuf)

    def _zrow(r, _):
        k = r // _S
        rr = r % _S
        for g in range(_D // 16):
            stage[k, rr, pl.ds(g * 16, 16)] = zeros
        stage[k, rr, pl.ds(_D - 16, 16)] = zeros
        return 0

    lax.fori_loop(0, 2 * _S, _zrow, 0)

    def _batch(brow, k, set_one):
        vs = [xbuf[brow, pl.ds(g * 16, 16)] for g in range(_SP // 16)]
        for s in range(_S):
            xv = vs[s // 16][s % 16]
            off = pl.multiple_of(xv & ~15, 16)
            if set_one:
                w = jnp.where(i16 == (xv & 15), 1.0, 0.0).astype(jnp.float32)
            else:
                w = zeros
            stage[k, s, pl.ds(off, 16)] = w

    for b in (0, 1):
        _batch(b, b, True)
        pltpu.async_copy(stage.at[b], out_hbm.at[base + b], sems[b])

    def _step(bp, _):
        for k in (0, 1):
            b = 2 * bp + k
            pltpu.make_async_copy(stage.at[k], out_hbm.at[base], sems[k]).wait()
            _batch(b - 2, k, False)
            _batch(b, k, True)
            pltpu.async_copy(stage.at[k], out_hbm.at[base + b], sems[k])
        return 0

    lax.fori_loop(1, _BPW // 2, _step, 0)

    pltpu.make_async_copy(stage.at[0], out_hbm.at[base], sems[0]).wait()
    pltpu.make_async_copy(stage.at[1], out_hbm.at[base], sems[1]).wait()


def kernel(x):
    x = x.astype(jnp.int32)
    xp = jnp.pad(x, ((0, 0), (0, _SP - _S)))
    mesh = plsc.VectorSubcoreMesh(
        core_axis_name="c", subcore_axis_name="s", num_cores=2
    )
    fn = functools.partial(
        pl.kernel,
        mesh=mesh,
        out_type=jax.ShapeDtypeStruct((_B, _S, _D), jnp.float32),
        scratch_types=[
            pltpu.VMEM((_BPW, _SP), jnp.int32),
            pltpu.VMEM((2, _S, _D), jnp.float32),
            pltpu.SemaphoreType.DMA,
            pltpu.SemaphoreType.DMA,
        ],
    )(_onehot_sc)
    return fn(xp)


# speedup vs baseline: 1.1076x; 1.0860x over previous
import jax
import jax.numpy as jnp
from jax import lax
from jax.experimental import pallas as pl

_DEPTH = 1000
_ROWS = 1024
_COLS = 50
_BLK = 64


def _onehot_body(x_ref, o_ref):
    xv = x_ref[...]
    col = lax.broadcasted_iota(jnp.int32, (_BLK, _COLS, _DEPTH), 2)
    o_ref[...] = (col == xv[:, :, None]).astype(jnp.float32)


def kernel(x):
    x = x.astype(jnp.int32)
    return pl.pallas_call(
        _onehot_body,
        grid=(_ROWS // _BLK,),
        in_specs=[pl.BlockSpec((_BLK, _COLS), lambda i: (i, 0))],
        out_specs=pl.BlockSpec((_BLK, _COLS, _DEPTH), lambda i: (i, 0, 0)),
        out_shape=jax.ShapeDtypeStruct((_ROWS, _COLS, _DEPTH), jnp.float32),
    )(x)


# speedup vs baseline: 4.9265x; 4.4480x over previous
import jax
import jax.numpy as jnp
from jax import lax
from jax.experimental import pallas as pl

_DEPTH = 1000
_ROWS = 1024
_COLS = 50
_BLK = 2


def _onehot_body(x_ref, o_ref):
    xv = x_ref[...]
    d = lax.broadcasted_iota(jnp.int32, (_BLK, _DEPTH, _ROWS), 1)
    o_ref[...] = (d == xv).astype(jnp.float32)


def kernel(x):
    xt = x.astype(jnp.int32).T.reshape(_COLS, 1, _ROWS)
    outt = pl.pallas_call(
        _onehot_body,
        grid=(_COLS // _BLK,),
        in_specs=[pl.BlockSpec((_BLK, 1, _ROWS), lambda i: (i, 0, 0))],
        out_specs=pl.BlockSpec((_BLK, _DEPTH, _ROWS), lambda i: (i, 0, 0)),
        out_shape=jax.ShapeDtypeStruct((_COLS, _DEPTH, _ROWS), jnp.float32),
    )(xt)
    return jnp.transpose(outt, (2, 0, 1))
